# baseline (speedup 1.0000x reference)
import jax
import jax.numpy as jnp
from jax import lax
from jax.experimental import pallas as pl

B, N, D = 2, 4096, 1024
H, DH = 16, 64
NQ, NKV = 512, 1024
HD = H * DH
NBLK = 512
NB = N // NBLK


def _cumsum_lanes(x):
    L = x.shape[1]
    s = 1
    while s < L:
        shifted = jnp.concatenate(
            [jnp.zeros((1, s), x.dtype), x[:, : L - s]], axis=1)
        x = x + shifted
        s *= 2
    return x


def _select_topk(logits, k):
    bits = lax.bitcast_convert_type(logits, jnp.int32)
    key = jnp.where(bits < 0, bits ^ jnp.int32(0x7FFFFFFF), bits)

    def body(_, carry):
        lo, hi = carry
        x = lo ^ hi
        mid = (lo & hi) + (x >> 1) + (x & 1)
        cnt = jnp.sum((key >= mid).astype(jnp.int32))
        ok = cnt >= k
        return jnp.where(ok, mid, lo), jnp.where(ok, hi, mid - 1)

    lo, _ = lax.fori_loop(
        0, 33, body, (jnp.int32(-2147483648), jnp.int32(2147483647)))
    tau = lo

    gt = (key > tau)
    tie = (key == tau)
    need = k - jnp.sum(gt.astype(jnp.int32))
    tie_cum = _cumsum_lanes(tie.astype(jnp.float32))
    sel_b = gt | (tie & (tie_cum <= need.astype(jnp.float32)))
    sel = sel_b.astype(jnp.float32)
    rank = (_cumsum_lanes(sel) - 1.0).astype(jnp.int32)
    return sel, rank


def _logits_kernel(x_ref, wq_ref, wkv_ref, ql_ref, kl_ref):
    x = x_ref[0]
    dn = (((1,), (1,)), ((), ()))
    ql_ref[0] = lax.dot_general(wq_ref[...], x, dn,
                                preferred_element_type=jnp.float32)
    kl_ref[0] = lax.dot_general(wkv_ref[...], x, dn,
                                preferred_element_type=jnp.float32)


def _topk_kernel(ql_ref, kl_ref,
                 qsel_ref, qrank_ref, qsig_ref,
                 kvsel_ref, kvrank_ref, kvsig_ref):
    ql = ql_ref[0]
    kl = kl_ref[0]
    qsel, qrank = _select_topk(ql, NQ)
    kvsel, kvrank = _select_topk(kl, NKV)
    qsel_ref[0] = qsel
    qrank_ref[0] = qrank
    qsig_ref[0] = jax.nn.sigmoid(ql)
    kvsel_ref[0] = kvsel
    kvrank_ref[0] = kvrank
    kvsig_ref[0] = jax.nn.sigmoid(kl)


def _onehot(rank, sel, rows):
    i = lax.broadcasted_iota(jnp.int32, (rows, rank.shape[1]), 0)
    return jnp.where((i == rank) & (sel > 0.5), 1.0, 0.0).astype(jnp.float32)


def _make_gather_kernel(rows):
    def _gather_kernel(x_ref, rank_ref, sel_ref, sig_ref, rot_ref,
                       g_ref, remb_ref, sc_ref):
        oh = _onehot(rank_ref[0, 0:1], sel_ref[0, 0:1], rows)
        part_g = jnp.dot(oh, x_ref[0], preferred_element_type=jnp.float32)
        part_r = jnp.dot(oh, rot_ref[...], preferred_element_type=jnp.float32)
        part_s = lax.dot_general(oh, sig_ref[0, 0:1],
                                 (((1,), (1,)), ((), ())),
                                 preferred_element_type=jnp.float32)

        @pl.when(pl.program_id(1) == 0)
        def _init():
            g_ref[0] = part_g
            remb_ref[0] = part_r
            sc_ref[0] = part_s

        @pl.when(pl.program_id(1) != 0)
        def _acc():
            g_ref[0] += part_g
            remb_ref[0] += part_r
            sc_ref[0] += part_s

    return _gather_kernel


def _layernorm(t, gamma):
    mu = jnp.mean(t, axis=1, keepdims=True)
    var = jnp.mean((t - mu) * (t - mu), axis=1, keepdims=True)
    return (t - mu) / jnp.sqrt(var + 1e-5) * gamma


def _rot_mats():
    r = lax.broadcasted_iota(jnp.int32, (HD, HD), 0)
    c = lax.broadcasted_iota(jnp.int32, (HD, HD), 1)
    cm = lax.rem(c, DH)
    p = jnp.where((r == c - DH // 2) & (cm >= DH // 2), 1.0, 0.0) \
        + jnp.where((r == c + DH // 2) & (cm < DH // 2), -1.0, 0.0)
    d = lax.broadcasted_iota(jnp.int32, (DH, HD), 0)
    cc = lax.broadcasted_iota(jnp.int32, (DH, HD), 1)
    tile = jnp.where(lax.rem(cc, DH) == d, 1.0, 0.0)
    return p.astype(jnp.float32), tile.astype(jnp.float32)


def _apply_rotary(t, remb):
    p, tile = _rot_mats()
    ct = jnp.dot(jnp.cos(remb), tile, preferred_element_type=jnp.float32)
    st = jnp.dot(jnp.sin(remb), tile, preferred_element_type=jnp.float32)
    rh = jnp.dot(t, p, preferred_element_type=jnp.float32)
    return t * ct + rh * st


def _proj_rot_kernel(g_ref, remb_ref, gamma_ref, w_ref, out_ref):
    tn = _layernorm(g_ref[0], gamma_ref[...])
    t = jnp.dot(tn, w_ref[...], preferred_element_type=jnp.float32)
    out_ref[0] = _apply_rotary(t, remb_ref[0])


def _proj_v_kernel(g_ref, sc_ref, gamma_ref, w_ref, out_ref):
    tn = _layernorm(g_ref[0], gamma_ref[...])
    t = jnp.dot(tn, w_ref[...], preferred_element_type=jnp.float32)
    out_ref[0] = t * sc_ref[0]


def _attn_kernel(q_ref, k_ref, v_ref, o_ref):
    q = q_ref[0, 0]
    k = k_ref[0, 0]
    v = v_ref[0, 0]
    sim = lax.dot_general(q, k, (((1,), (1,)), ((), ())),
                          preferred_element_type=jnp.float32)
    sim = sim * (DH ** -0.5)
    m = jnp.max(sim, axis=1, keepdims=True)
    e = jnp.exp(sim - m)
    a = e / jnp.sum(e, axis=1, keepdims=True)
    o_ref[0, 0] = jnp.dot(a, v, preferred_element_type=jnp.float32)


def _proj_o_kernel(ao_ref, wo_ref, qs_ref, o_ref):
    o = jnp.dot(ao_ref[0], wo_ref[...], preferred_element_type=jnp.float32)
    o_ref[0] = o * qs_ref[0]


def _scatter_kernel(o_ref, rank_ref, sel_ref, null_ref, out_ref):
    oh = _onehot(rank_ref[0, 0:1], sel_ref[0, 0:1], NQ)
    scat = lax.dot_general(oh, o_ref[0], (((0,), (0,)), ((), ())),
                           preferred_element_type=jnp.float32)
    selc = lax.dot_general(oh, jnp.ones((NQ, 1), jnp.float32),
                           (((0,), (0,)), ((), ())),
                           preferred_element_type=jnp.float32)
    out_ref[0] = scat + (1.0 - selc) * null_ref[...]


def _row_spec():
    return pl.BlockSpec((1, 1, N), lambda b: (b, 0, 0))


def _row_blk_spec():
    return pl.BlockSpec((1, 1, NBLK), lambda b, nb: (b, 0, nb))


def _full_spec(shape):
    return pl.BlockSpec(shape, lambda *_: (0,) * len(shape))


def _gather(x, rank, sel, sig, rotary_emb, rows):
    f32 = jnp.float32
    return pl.pallas_call(
        _make_gather_kernel(rows),
        grid=(B, NB),
        in_specs=[
            pl.BlockSpec((1, NBLK, D), lambda b, nb: (b, nb, 0)),
            _row_blk_spec(), _row_blk_spec(), _row_blk_spec(),
            pl.BlockSpec((NBLK, DH), lambda b, nb: (nb, 0)),
        ],
        out_specs=[
            pl.BlockSpec((1, rows, D), lambda b, nb: (b, 0, 0)),
            pl.BlockSpec((1, rows, DH), lambda b, nb: (b, 0, 0)),
            pl.BlockSpec((1, rows, 1), lambda b, nb: (b, 0, 0)),
        ],
        out_shape=[
            jax.ShapeDtypeStruct((B, rows, D), f32),
            jax.ShapeDtypeStruct((B, rows, DH), f32),
            jax.ShapeDtypeStruct((B, rows, 1), f32),
        ],
    )(x, rank, sel, sig, rotary_emb)


def _proj_rot(g, remb, g2, w, rows):
    return pl.pallas_call(
        _proj_rot_kernel,
        grid=(B,),
        in_specs=[
            pl.BlockSpec((1, rows, D), lambda b: (b, 0, 0)),
            pl.BlockSpec((1, rows, DH), lambda b: (b, 0, 0)),
            _full_spec((1, D)),
            _full_spec((D, HD)),
        ],
        out_specs=pl.BlockSpec((1, rows, HD), lambda b: (b, 0, 0)),
        out_shape=jax.ShapeDtypeStruct((B, rows, HD), jnp.float32),
    )(g, remb, g2, w)


@jax.jit
def kernel(x, rotary_emb, w_q_router, w_kv_router, ln_gamma, Wq, Wk, Wv, Wo,
           null_tokens):
    f32 = jnp.float32
    wq2 = w_q_router.reshape(1, D)
    wkv2 = w_kv_router.reshape(1, D)
    g2 = ln_gamma.reshape(1, D)
    null2 = null_tokens.reshape(1, D)

    ql, kl = pl.pallas_call(
        _logits_kernel,
        grid=(B, NB),
        in_specs=[
            pl.BlockSpec((1, NBLK, D), lambda b, nb: (b, nb, 0)),
            _full_spec((1, D)),
            _full_spec((1, D)),
        ],
        out_specs=[_row_blk_spec(), _row_blk_spec()],
        out_shape=[jax.ShapeDtypeStruct((B, 1, N), f32)] * 2,
    )(x, wq2, wkv2)

    row_f = jax.ShapeDtypeStruct((B, 1, N), f32)
    row_i = jax.ShapeDtypeStruct((B, 1, N), jnp.int32)
    qsel, qrank, qsig, kvsel, kvrank, kvsig = pl.pallas_call(
        _topk_kernel,
        grid=(B,),
        in_specs=[_row_spec(), _row_spec()],
        out_specs=[_row_spec()] * 6,
        out_shape=[row_f, row_i, row_f, row_f, row_i, row_f],
    )(ql, kl)

    qg, qremb, qs = _gather(x, qrank, qsel, qsig, rotary_emb, NQ)
    kvg, kvremb, ks = _gather(x, kvrank, kvsel, kvsig, rotary_emb, NKV)

    q = _proj_rot(qg, qremb, g2, Wq, NQ)
    k = _proj_rot(kvg, kvremb, g2, Wk, NKV)
    v = pl.pallas_call(
        _proj_v_kernel,
        grid=(B,),
        in_specs=[
            pl.BlockSpec((1, NKV, D), lambda b: (b, 0, 0)),
            pl.BlockSpec((1, NKV, 1), lambda b: (b, 0, 0)),
            _full_spec((1, D)),
            _full_spec((D, HD)),
        ],
        out_specs=pl.BlockSpec((1, NKV, HD), lambda b: (b, 0, 0)),
        out_shape=jax.ShapeDtypeStruct((B, NKV, HD), f32),
    )(kvg, ks, g2, Wv)

    q4 = q.reshape(B, NQ, H, DH).transpose(0, 2, 1, 3)
    k4 = k.reshape(B, NKV, H, DH).transpose(0, 2, 1, 3)
    v4 = v.reshape(B, NKV, H, DH).transpose(0, 2, 1, 3)
    ao4 = pl.pallas_call(
        _attn_kernel,
        grid=(B, H),
        in_specs=[
            pl.BlockSpec((1, 1, NQ, DH), lambda b, h: (b, h, 0, 0)),
            pl.BlockSpec((1, 1, NKV, DH), lambda b, h: (b, h, 0, 0)),
            pl.BlockSpec((1, 1, NKV, DH), lambda b, h: (b, h, 0, 0)),
        ],
        out_specs=pl.BlockSpec((1, 1, NQ, DH), lambda b, h: (b, h, 0, 0)),
        out_shape=jax.ShapeDtypeStruct((B, H, NQ, DH), f32),
    )(q4, k4, v4)
    ao = ao4.transpose(0, 2, 1, 3).reshape(B, NQ, HD)

    o = pl.pallas_call(
        _proj_o_kernel,
        grid=(B,),
        in_specs=[
            pl.BlockSpec((1, NQ, HD), lambda b: (b, 0, 0)),
            _full_spec((HD, D)),
            pl.BlockSpec((1, NQ, 1), lambda b: (b, 0, 0)),
        ],
        out_specs=pl.BlockSpec((1, NQ, D), lambda b: (b, 0, 0)),
        out_shape=jax.ShapeDtypeStruct((B, NQ, D), f32),
    )(ao, Wo, qs)

    result = pl.pallas_call(
        _scatter_kernel,
        grid=(B, NB),
        in_specs=[
            pl.BlockSpec((1, NQ, D), lambda b, nb: (b, 0, 0)),
            _row_blk_spec(), _row_blk_spec(),
            _full_spec((1, D)),
        ],
        out_specs=pl.BlockSpec((1, NBLK, D), lambda b, nb: (b, nb, 0)),
        out_shape=jax.ShapeDtypeStruct((B, N, D), f32),
    )(o, qrank, qsel, null2)

    return result
